# baseline (speedup 1.0000x reference)
import functools

import jax
import jax.numpy as jnp
from jax import lax
from jax.experimental import pallas as pl
from jax.experimental.pallas import tpu as pltpu
from jax.experimental.pallas import tpu_sc as plsc

_D = 32
_B1 = 4096
_B2 = 200
_B = _B1 * _B2
_SCALE = float(_D) ** 0.5

_info = plsc.get_sparse_core_info()
_NC, _NS, _L = _info.num_cores, _info.num_subcores, _info.num_lanes
_NW = _NC * _NS

_GRP = 128
_TOK = 512
_GPB = _TOK // _GRP
_BLK_PER_ROW = _B1 // _TOK
_NBLK = _B // _TOK
_BPW = _NBLK // _NW
_NP = _BPW // 2

_mesh = plsc.VectorSubcoreMesh(core_axis_name="c", subcore_axis_name="s")


@functools.partial(
    pl.kernel,
    mesh=_mesh,
    out_type=jax.ShapeDtypeStruct((_B2, _D, _B1), jnp.float32),
    scratch_types=[
        pltpu.VMEM((_GPB, _GRP), jnp.int32),
        pltpu.VMEM((_GPB, _GRP), jnp.int32),
        pltpu.VMEM((_TOK, _D), jnp.float32),
        pltpu.VMEM((_TOK, _D), jnp.float32),
        pltpu.VMEM((_D, _TOK), jnp.float32),
        pltpu.VMEM((_D, _TOK), jnp.float32),
        pltpu.SemaphoreType.DMA,
        pltpu.SemaphoreType.DMA,
        pltpu.SemaphoreType.DMA,
        pltpu.SemaphoreType.DMA,
        pltpu.SemaphoreType.DMA,
    ],
    compiler_params=pltpu.CompilerParams(use_tc_tiling_on_sc=False,
                                         needs_layout_passes=False),
)
def _emb_lookup(table_hbm, x3_hbm, out_hbm, idx_a, idx_b, rows_a, rows_b,
                tbuf_a, tbuf_b, gsem_a, gsem_b, isem, osem_a, osem_b):
    wid = lax.axis_index("s") * _NC + lax.axis_index("c")
    first = wid * _BPW

    def loc(t):
        f = first + t
        return f // _BLK_PER_ROW, lax.rem(f, _BLK_PER_ROW)

    def idx_copy(t, idx_v):
        b2, bb = loc(t)
        return pltpu.make_async_copy(
            x3_hbm.at[b2, pl.ds(bb * _GPB, _GPB)], idx_v, isem)

    def gather_descs(idx_v, rows_v, gsem):
        return [
            pltpu.make_async_copy(
                table_hbm.at[idx_v.at[j]],
                rows_v.at[pl.ds(j * _GRP, _GRP)],
                gsem,
            )
            for j in range(_GPB)
        ]

    def out_desc(t, tbuf, osem):
        b2, bb = loc(t)
        return pltpu.make_async_copy(
            tbuf, out_hbm.at[b2, :, pl.ds(bb * _TOK, _TOK)], osem)

    lane = jnp.arange(_L, dtype=jnp.int32)
    dconst = [jnp.full((_L,), d, jnp.int32) for d in range(_D)]

    def compute(rows_v, tbuf):
        def body(g, c):
            tokvec = lane + g * _L
            for d in range(_D):
                v = plsc.load_gather(rows_v, [tokvec, dconst[d]]) * _SCALE
                tbuf[d, pl.ds(g * _L, _L)] = v
            return c

        lax.fori_loop(0, _TOK // _L, body, 0)

    idx_copy(0, idx_a).start()
    idx_copy(0, idx_a).wait()
    for desc in gather_descs(idx_a, rows_a, gsem_a):
        desc.start()
    idx_copy(1, idx_b).start()
    idx_copy(1, idx_b).wait()
    for desc in gather_descs(idx_b, rows_b, gsem_b):
        desc.start()

    def half(p, t, idx_v, rows_v, tbuf, gsem, osem):
        for desc in gather_descs(idx_v, rows_v, gsem):
            desc.wait()

        @pl.when(p + 1 < _NP)
        def _():
            idx_copy(t + 2, idx_v).start()

        @pl.when(p >= 1)
        def _():
            out_desc(t - 2, tbuf, osem).wait()

        compute(rows_v, tbuf)
        out_desc(t, tbuf, osem).start()

        @pl.when(p + 1 < _NP)
        def _():
            idx_copy(t + 2, idx_v).wait()
            for desc in gather_descs(idx_v, rows_v, gsem):
                desc.start()

    def pair_body(p, carry):
        a = p * 2
        half(p, a, idx_a, rows_a, tbuf_a, gsem_a, osem_a)
        half(p, a + 1, idx_b, rows_b, tbuf_b, gsem_b, osem_b)
        return carry

    lax.fori_loop(0, _NP, pair_body, 0)

    out_desc(_BPW - 2, tbuf_a, osem_a).wait()
    out_desc(_BPW - 1, tbuf_b, osem_b).wait()


def kernel(x, table):
    x3 = jnp.transpose(x).reshape(_B2, _B1 // _GRP, _GRP).astype(jnp.int32)
    out_t = _emb_lookup(table, x3)
    return jnp.transpose(out_t, (2, 0, 1))


# speedup vs baseline: 1.3620x; 1.3620x over previous
import functools

import jax
import jax.numpy as jnp
from jax import lax
from jax.experimental import pallas as pl
from jax.experimental.pallas import tpu as pltpu
from jax.experimental.pallas import tpu_sc as plsc

_D = 32
_B1 = 4096
_B2 = 200
_B = _B1 * _B2
_SCALE = float(_D) ** 0.5

_info = plsc.get_sparse_core_info()
_NC, _NS, _L = _info.num_cores, _info.num_subcores, _info.num_lanes
_NW = _NC * _NS

_GRP = 128
_TOK = 512
_GPB = _TOK // _GRP
_BLK_PER_ROW = _B1 // _TOK
_NBLK = _B // _TOK
_BPW = _NBLK // _NW
_NP = _BPW // 2

_mesh = plsc.VectorSubcoreMesh(core_axis_name="c", subcore_axis_name="s")


@functools.partial(
    pl.kernel,
    mesh=_mesh,
    out_type=jax.ShapeDtypeStruct((_NBLK, _TOK, _D), jnp.float32),
    scratch_types=[
        pltpu.VMEM((_GPB, _GRP), jnp.int32),
        pltpu.VMEM((_GPB, _GRP), jnp.int32),
        pltpu.VMEM((_TOK, _D), jnp.float32),
        pltpu.VMEM((_TOK, _D), jnp.float32),
        pltpu.VMEM((_TOK, _D), jnp.float32),
        pltpu.VMEM((_TOK, _D), jnp.float32),
        pltpu.SemaphoreType.DMA,
        pltpu.SemaphoreType.DMA,
        pltpu.SemaphoreType.DMA,
        pltpu.SemaphoreType.DMA,
        pltpu.SemaphoreType.DMA,
    ],
    compiler_params=pltpu.CompilerParams(use_tc_tiling_on_sc=False,
                                         needs_layout_passes=False),
)
def _emb_lookup(table_hbm, x3_hbm, out_hbm, idx_a, idx_b, rows_a, rows_b,
                tbuf_a, tbuf_b, gsem_a, gsem_b, isem, osem_a, osem_b):
    wid = lax.axis_index("s") * _NC + lax.axis_index("c")
    first = wid * _BPW

    def loc(t):
        f = first + t
        return f // _BLK_PER_ROW, lax.rem(f, _BLK_PER_ROW)

    def idx_copy(t, idx_v):
        b2, bb = loc(t)
        return pltpu.make_async_copy(
            x3_hbm.at[b2, pl.ds(bb * _GPB, _GPB)], idx_v, isem)

    def gather_descs(idx_v, rows_v, gsem):
        return [
            pltpu.make_async_copy(
                table_hbm.at[idx_v.at[j]],
                rows_v.at[pl.ds(j * _GRP, _GRP)],
                gsem,
            )
            for j in range(_GPB)
        ]

    def out_desc(t, tbuf, osem):
        b2, bb = loc(t)
        return pltpu.make_async_copy(
            tbuf, out_hbm.at[b2 * _BLK_PER_ROW + bb], osem)

    def compute(rows_v, tbuf):
        def body(r, c):
            tok = r * 4
            for u in range(4):
                for h in range(2):
                    sl = pl.ds(h * _L, _L)
                    tbuf[tok + u, sl] = rows_v[tok + u, sl] * _SCALE
            return c

        lax.fori_loop(0, _TOK // 4, body, 0)

    idx_copy(0, idx_a).start()
    idx_copy(0, idx_a).wait()
    for desc in gather_descs(idx_a, rows_a, gsem_a):
        desc.start()
    idx_copy(1, idx_b).start()
    idx_copy(1, idx_b).wait()
    for desc in gather_descs(idx_b, rows_b, gsem_b):
        desc.start()

    def half(p, t, idx_v, rows_v, tbuf, gsem, osem):
        for desc in gather_descs(idx_v, rows_v, gsem):
            desc.wait()

        @pl.when(p + 1 < _NP)
        def _():
            idx_copy(t + 2, idx_v).start()

        @pl.when(p >= 1)
        def _():
            out_desc(t - 2, tbuf, osem).wait()

        compute(rows_v, tbuf)
        out_desc(t, tbuf, osem).start()

        @pl.when(p + 1 < _NP)
        def _():
            idx_copy(t + 2, idx_v).wait()
            for desc in gather_descs(idx_v, rows_v, gsem):
                desc.start()

    def pair_body(p, carry):
        a = p * 2
        half(p, a, idx_a, rows_a, tbuf_a, gsem_a, osem_a)
        half(p, a + 1, idx_b, rows_b, tbuf_b, gsem_b, osem_b)
        return carry

    lax.fori_loop(0, _NP, pair_body, 0)

    out_desc(_BPW - 2, tbuf_a, osem_a).wait()
    out_desc(_BPW - 1, tbuf_b, osem_b).wait()


def kernel(x, table):
    x3 = jnp.transpose(x).reshape(_B2, _B1 // _GRP, _GRP).astype(jnp.int32)
    out_t = _emb_lookup(table, x3).reshape(_B2, _B1, _D)
    return jnp.transpose(out_t, (1, 0, 2))


# speedup vs baseline: 1.4880x; 1.0925x over previous
import functools

import jax
import jax.numpy as jnp
from jax import lax
from jax.experimental import pallas as pl
from jax.experimental.pallas import tpu as pltpu
from jax.experimental.pallas import tpu_sc as plsc

_D = 32
_B1 = 4096
_B2 = 200
_B = _B1 * _B2
_SCALE = float(_D) ** 0.5

_info = plsc.get_sparse_core_info()
_NC, _NS, _L = _info.num_cores, _info.num_subcores, _info.num_lanes
_NW = _NC * _NS

_GRP = 128
_TOK = 512
_GPB = _TOK // _GRP
_BLK_PER_ROW = _B1 // _TOK
_NBLK = _B // _TOK
_BPW = _NBLK // _NW
_NP = _BPW // 2

_mesh = plsc.VectorSubcoreMesh(core_axis_name="c", subcore_axis_name="s")


@functools.partial(
    pl.kernel,
    mesh=_mesh,
    out_type=jax.ShapeDtypeStruct((_B2, _D, _B1), jnp.float32),
    scratch_types=[
        pltpu.VMEM((_GPB, _GRP), jnp.int32),
        pltpu.VMEM((_GPB, _GRP), jnp.int32),
        pltpu.VMEM((_TOK, _D), jnp.float32),
        pltpu.VMEM((_TOK, _D), jnp.float32),
        pltpu.VMEM((_D, _TOK), jnp.float32),
        pltpu.VMEM((_D, _TOK), jnp.float32),
        pltpu.SemaphoreType.DMA,
        pltpu.SemaphoreType.DMA,
        pltpu.SemaphoreType.DMA,
        pltpu.SemaphoreType.DMA,
        pltpu.SemaphoreType.DMA,
    ],
    compiler_params=pltpu.CompilerParams(use_tc_tiling_on_sc=False,
                                         needs_layout_passes=False),
)
def _emb_lookup(table_hbm, x3_hbm, out_hbm, idx_a, idx_b, rows_a, rows_b,
                tbuf_a, tbuf_b, gsem_a, gsem_b, isem, osem_a, osem_b):
    wid = lax.axis_index("s") * _NC + lax.axis_index("c")
    first = wid * _BPW

    def loc(t):
        f = first + t
        return f // _BLK_PER_ROW, lax.rem(f, _BLK_PER_ROW)

    def idx_copy(t, idx_v):
        b2, bb = loc(t)
        return pltpu.make_async_copy(
            x3_hbm.at[b2, pl.ds(bb * _GPB, _GPB)], idx_v, isem)

    def gather_descs(idx_v, rows_v, gsem):
        return [
            pltpu.make_async_copy(
                table_hbm.at[idx_v.at[j]],
                rows_v.at[pl.ds(j * _GRP, _GRP)],
                gsem,
            )
            for j in range(_GPB)
        ]

    def out_desc(t, tbuf, osem):
        b2, bb = loc(t)
        return pltpu.make_async_copy(
            tbuf, out_hbm.at[b2, :, pl.ds(bb * _TOK, _TOK)], osem)

    lane = jnp.arange(_L, dtype=jnp.int32)
    dskew = [[(h * _L + ((lane + s) & (_L - 1))).astype(jnp.int32)
              for s in range(_L)] for h in range(2)]

    def compute(rows_v, tbuf):
        def body(tt, c):
            tokvec = lane + tt * _L
            for h in range(2):
                for s in range(_L):
                    v = plsc.load_gather(rows_v, [tokvec, dskew[h][s]])
                    plsc.store_scatter(tbuf, [dskew[h][s], tokvec],
                                       v * _SCALE)
            return c

        lax.fori_loop(0, _TOK // _L, body, 0)

    idx_copy(0, idx_a).start()
    idx_copy(0, idx_a).wait()
    for desc in gather_descs(idx_a, rows_a, gsem_a):
        desc.start()
    idx_copy(1, idx_b).start()
    idx_copy(1, idx_b).wait()
    for desc in gather_descs(idx_b, rows_b, gsem_b):
        desc.start()

    def half(p, t, idx_v, rows_v, tbuf, gsem, osem):
        for desc in gather_descs(idx_v, rows_v, gsem):
            desc.wait()

        @pl.when(p + 1 < _NP)
        def _():
            idx_copy(t + 2, idx_v).start()

        @pl.when(p >= 1)
        def _():
            out_desc(t - 2, tbuf, osem).wait()

        compute(rows_v, tbuf)
        out_desc(t, tbuf, osem).start()

        @pl.when(p + 1 < _NP)
        def _():
            idx_copy(t + 2, idx_v).wait()
            for desc in gather_descs(idx_v, rows_v, gsem):
                desc.start()

    def pair_body(p, carry):
        a = p * 2
        half(p, a, idx_a, rows_a, tbuf_a, gsem_a, osem_a)
        half(p, a + 1, idx_b, rows_b, tbuf_b, gsem_b, osem_b)
        return carry

    lax.fori_loop(0, _NP, pair_body, 0)

    out_desc(_BPW - 2, tbuf_a, osem_a).wait()
    out_desc(_BPW - 1, tbuf_b, osem_b).wait()


def kernel(x, table):
    x3 = jnp.transpose(x).reshape(_B2, _B1 // _GRP, _GRP).astype(jnp.int32)
    out_t = _emb_lookup(table, x3)
    return jnp.transpose(out_t, (2, 0, 1))


# speedup vs baseline: 1.7900x; 1.2029x over previous
import functools

import jax
import jax.numpy as jnp
from jax import lax
from jax.experimental import pallas as pl
from jax.experimental.pallas import tpu as pltpu
from jax.experimental.pallas import tpu_sc as plsc

_D = 32
_B1 = 4096
_B2 = 200
_B = _B1 * _B2
_SCALE = float(_D) ** 0.5

_info = plsc.get_sparse_core_info()
_NC, _NS, _L = _info.num_cores, _info.num_subcores, _info.num_lanes
_NW = _NC * _NS

_GRP = 128
_TOK = 512
_GPB = _TOK // _GRP
_BLK_PER_ROW = _B1 // _TOK
_NBLK = _B // _TOK
_BPW = _NBLK // _NW
_NP = _BPW // 2

_mesh = plsc.VectorSubcoreMesh(core_axis_name="c", subcore_axis_name="s")


@functools.partial(
    pl.kernel,
    mesh=_mesh,
    out_type=jax.ShapeDtypeStruct((_B2, _D, _B1), jnp.float32),
    scratch_types=[
        pltpu.VMEM((_GPB, _GRP), jnp.int32),
        pltpu.VMEM((_GPB, _GRP), jnp.int32),
        pltpu.VMEM((_TOK, _D), jnp.float32),
        pltpu.VMEM((_TOK, _D), jnp.float32),
        pltpu.VMEM((_D, _TOK), jnp.float32),
        pltpu.VMEM((_D, _TOK), jnp.float32),
        pltpu.SemaphoreType.DMA,
        pltpu.SemaphoreType.DMA,
        pltpu.SemaphoreType.DMA,
        pltpu.SemaphoreType.DMA,
        pltpu.SemaphoreType.DMA,
    ],
    compiler_params=pltpu.CompilerParams(use_tc_tiling_on_sc=False,
                                         needs_layout_passes=False),
)
def _emb_lookup(table_hbm, x3_hbm, out_hbm, idx_a, idx_b, rows_a, rows_b,
                tbuf_a, tbuf_b, gsem_a, gsem_b, isem, osem_a, osem_b):
    wid = lax.axis_index("s") * _NC + lax.axis_index("c")
    first = wid * _BPW

    def loc(t):
        f = first + t
        return f // _BLK_PER_ROW, lax.rem(f, _BLK_PER_ROW)

    def idx_copy(t, idx_v):
        b2, bb = loc(t)
        return pltpu.make_async_copy(
            x3_hbm.at[b2, pl.ds(bb * _GPB, _GPB)], idx_v, isem)

    def gather_descs(idx_v, rows_v, gsem):
        return [
            pltpu.make_async_copy(
                table_hbm.at[idx_v.at[j]],
                rows_v.at[pl.ds(j * _GRP, _GRP)],
                gsem,
            )
            for j in range(_GPB)
        ]

    def out_desc(t, tbuf, osem):
        b2, bb = loc(t)
        return pltpu.make_async_copy(
            tbuf, out_hbm.at[b2, :, pl.ds(bb * _TOK, _TOK)], osem)

    lane = jnp.arange(_L, dtype=jnp.int32)
    dskew = [[(h * _L + ((lane + s) & (_L - 1))).astype(jnp.int32)
              for s in range(_L)] for h in range(2)]

    def compute(rows_v, tbuf):
        def body(tt, c):
            tokvec = lane + tt * _L
            vals = [
                plsc.load_gather(rows_v, [tokvec, dskew[h][s]]) * _SCALE
                for h in range(2) for s in range(_L)
            ]
            for (h, s), v in zip(
                    [(h, s) for h in range(2) for s in range(_L)], vals):
                plsc.store_scatter(tbuf, [dskew[h][s], tokvec], v)
            return c

        lax.fori_loop(0, _TOK // _L, body, 0)

    idx_copy(0, idx_a).start()
    idx_copy(0, idx_a).wait()
    for desc in gather_descs(idx_a, rows_a, gsem_a):
        desc.start()
    idx_copy(1, idx_b).start()
    idx_copy(1, idx_b).wait()
    for desc in gather_descs(idx_b, rows_b, gsem_b):
        desc.start()

    def half(p, t, idx_v, rows_v, tbuf, gsem, osem):
        for desc in gather_descs(idx_v, rows_v, gsem):
            desc.wait()

        @pl.when(p + 1 < _NP)
        def _():
            idx_copy(t + 2, idx_v).start()

        @pl.when(p >= 1)
        def _():
            out_desc(t - 2, tbuf, osem).wait()

        compute(rows_v, tbuf)
        out_desc(t, tbuf, osem).start()

        @pl.when(p + 1 < _NP)
        def _():
            idx_copy(t + 2, idx_v).wait()
            for desc in gather_descs(idx_v, rows_v, gsem):
                desc.start()

    def pair_body(p, carry):
        a = p * 2
        half(p, a, idx_a, rows_a, tbuf_a, gsem_a, osem_a)
        half(p, a + 1, idx_b, rows_b, tbuf_b, gsem_b, osem_b)
        return carry

    lax.fori_loop(0, _NP, pair_body, 0)

    out_desc(_BPW - 2, tbuf_a, osem_a).wait()
    out_desc(_BPW - 1, tbuf_b, osem_b).wait()


def kernel(x, table):
    x3 = jnp.transpose(x).reshape(_B2, _B1 // _GRP, _GRP).astype(jnp.int32)
    out_t = _emb_lookup(table, x3)
    return jnp.transpose(out_t, (2, 0, 1))
